# Initial kernel scaffold; baseline (speedup 1.0000x reference)
import jax
import jax.numpy as jnp
from jax.experimental import pallas as pl


def kernel(x, index):
    raise NotImplementedError("write your pallas kernel here")


# speedup vs baseline: 1.7397x; 1.7397x over previous
import functools
import jax
import jax.numpy as jnp
from jax import lax
from jax.experimental import pallas as pl
from jax.experimental.pallas import tpu as pltpu
from jax.experimental.pallas import tpu_sc as plsc

NC = 2
NS = 16
NW = NC * NS

B, C, H, W = 16, 192, 56, 56
G = 4
ROWS = B * C
ROW = H * W
RPW = ROWS // NW
K = 8
NCHUNK = RPW // K
LANES = 16
VPR = ROW // LANES


def _body(x_hbm, ridx_hbm, out_hbm, idx_v, rows_v, out_v, sem):
    c = lax.axis_index("c")
    s = lax.axis_index("s")
    w = s * NC + c
    base = w * RPW
    pltpu.sync_copy(ridx_hbm.at[pl.ds(w * RPW * G, RPW * G)], idx_v)

    def chunk(ci, carry):
        pltpu.async_copy(
            x_hbm.at[idx_v.at[pl.ds(ci * (K * G), K * G)]], rows_v, sem
        ).wait()

        def vloop(i, carry2):
            off = i * LANES
            for k in range(K):
                a = jnp.maximum(
                    rows_v[G * k, pl.ds(off, LANES)],
                    rows_v[G * k + 1, pl.ds(off, LANES)],
                )
                b2 = jnp.maximum(
                    rows_v[G * k + 2, pl.ds(off, LANES)],
                    rows_v[G * k + 3, pl.ds(off, LANES)],
                )
                out_v[k, pl.ds(off, LANES)] = jnp.maximum(a, b2)
            return carry2

        lax.fori_loop(0, VPR, vloop, 0)
        pltpu.sync_copy(out_v, out_hbm.at[pl.ds(base + ci * K, K)])
        return carry

    lax.fori_loop(0, NCHUNK, chunk, 0)


@functools.partial(jax.jit)
def _run(xr, rid):
    mesh = plsc.VectorSubcoreMesh(core_axis_name="c", subcore_axis_name="s")
    f = functools.partial(
        pl.kernel,
        out_type=jax.ShapeDtypeStruct((ROWS, ROW), jnp.float32),
        mesh=mesh,
        compiler_params=pltpu.CompilerParams(use_tc_tiling_on_sc=False),
        scratch_types=[
            pltpu.VMEM((RPW * G,), jnp.int32),
            pltpu.VMEM((K * G, ROW), jnp.float32),
            pltpu.VMEM((K, ROW), jnp.float32),
            pltpu.SemaphoreType.DMA,
        ],
    )(_body)
    return f(xr, rid)


def kernel(x, index):
    idx4 = index.astype(jnp.int32) % C
    rid = (
        jnp.arange(B, dtype=jnp.int32)[:, None] * C + idx4[None, :]
    ).reshape(-1)
    xr = x.reshape(ROWS, ROW)
    outr = _run(xr, rid)
    return outr.reshape(B, C, H, W)


# speedup vs baseline: 1.9174x; 1.1022x over previous
import functools
import jax
import jax.numpy as jnp
from jax import lax
from jax.experimental import pallas as pl
from jax.experimental.pallas import tpu as pltpu
from jax.experimental.pallas import tpu_sc as plsc

NC = 2
NS = 16
NW = NC * NS

B, C, H, W = 16, 192, 56, 56
G = 4
ROWS = B * C
ROW = H * W
ROWP = 3200
RPW = ROWS // NW
K = 4
NCHUNK = RPW // K
LANES = 16
VPR = ROWP // LANES


def _compute_chunk(rows_v, out_v, ko):
    def vloop(i, carry):
        off = i * LANES
        for k in range(K):
            a = jnp.maximum(
                rows_v[G * k, pl.ds(off, LANES)],
                rows_v[G * k + 1, pl.ds(off, LANES)],
            )
            b2 = jnp.maximum(
                rows_v[G * k + 2, pl.ds(off, LANES)],
                rows_v[G * k + 3, pl.ds(off, LANES)],
            )
            out_v[ko + k, pl.ds(off, LANES)] = jnp.maximum(a, b2)
        return carry

    lax.fori_loop(0, VPR, vloop, 0)


def _body(x_hbm, ridx_hbm, out_hbm, idx_v, rows_v0, rows_v1, out_v, sem0, sem1):
    c = lax.axis_index("c")
    s = lax.axis_index("s")
    w = s * NC + c
    base = w * RPW
    pltpu.sync_copy(ridx_hbm.at[pl.ds(w * RPW * G, RPW * G)], idx_v)

    bufs = (rows_v0, rows_v1)
    sems = (sem0, sem1)

    def gather_start(ci, buf, sem):
        pltpu.async_copy(
            x_hbm.at[idx_v.at[pl.ds(ci * (K * G), K * G)]], buf, sem
        )

    def gather_wait(buf, sem):
        pltpu.make_async_copy(x_hbm.at[idx_v.at[pl.ds(0, K * G)]], buf, sem).wait()

    gather_start(0, rows_v0, sem0)
    gather_start(1, rows_v1, sem1)

    def pair(g, carry):
        ci0 = g * 2
        for b in range(2):
            ci = ci0 + b
            gather_wait(bufs[b], sems[b])
            _compute_chunk(bufs[b], out_v, b * K)

            @pl.when(ci + 2 < NCHUNK)
            def _():
                gather_start(ci + 2, bufs[b], sems[b])

        pltpu.sync_copy(out_v, out_hbm.at[pl.ds(base + ci0 * K, 2 * K)])
        return carry

    lax.fori_loop(0, NCHUNK // 2, pair, 0)


@jax.jit
def _run(xp, rid):
    mesh = plsc.VectorSubcoreMesh(core_axis_name="c", subcore_axis_name="s")
    f = functools.partial(
        pl.kernel,
        out_type=jax.ShapeDtypeStruct((ROWS, ROWP), jnp.float32),
        mesh=mesh,
        scratch_types=[
            pltpu.VMEM((RPW * G,), jnp.int32),
            pltpu.VMEM((K * G, ROWP), jnp.float32),
            pltpu.VMEM((K * G, ROWP), jnp.float32),
            pltpu.VMEM((2 * K, ROWP), jnp.float32),
            pltpu.SemaphoreType.DMA,
            pltpu.SemaphoreType.DMA,
        ],
    )(_body)
    return f(xp, rid)


def kernel(x, index):
    idx4 = index.astype(jnp.int32) % C
    rid = (
        jnp.arange(B, dtype=jnp.int32)[:, None] * C + idx4[None, :]
    ).reshape(-1)
    xp = jnp.pad(x.reshape(ROWS, ROW), ((0, 0), (0, ROWP - ROW)))
    outp = _run(xp, rid)
    return outp[:, :ROW].reshape(B, C, H, W)
